# baseline (speedup 1.0000x reference)
import functools

import jax
import jax.numpy as jnp
from jax import lax
from jax.experimental import pallas as pl
from jax.experimental.pallas import tpu as pltpu
from jax.experimental.pallas import tpu_sc as plsc

NUM_FACTORS = 32
BATCH = 16384
REG = 1e-07

NC, NS, L = 2, 16, 16
NW = NC * NS
BPW = BATCH // NW
ICHUNK = 128
NCHUNK = BPW // ICHUNK
NGROUPS = BPW // L

_mesh = plsc.VectorSubcoreMesh(
    core_axis_name="c", subcore_axis_name="s", num_cores=NC, num_subcores=NS
)


@functools.partial(
    pl.kernel,
    out_type=(
        jax.ShapeDtypeStruct((NW, BPW), jnp.float32),
        jax.ShapeDtypeStruct((NW, L), jnp.float32),
    ),
    mesh=_mesh,
    scratch_types=[
        pltpu.VMEM((NCHUNK, ICHUNK), jnp.int32),
        pltpu.VMEM((NCHUNK, ICHUNK), jnp.int32),
        pltpu.VMEM((NCHUNK, ICHUNK), jnp.int32),
        pltpu.VMEM((BPW, NUM_FACTORS), jnp.float32),
        pltpu.VMEM((BPW, NUM_FACTORS), jnp.float32),
        pltpu.VMEM((BPW, NUM_FACTORS), jnp.float32),
        pltpu.VMEM((BPW,), jnp.float32),
        pltpu.VMEM((L,), jnp.float32),
        pltpu.SemaphoreType.DMA,
    ],
    compiler_params=pltpu.CompilerParams(
        needs_layout_passes=False, use_tc_tiling_on_sc=False
    ),
)
def _sc_bpr(bond_hbm, bett_hbm, wors_hbm, table_hbm, diff_hbm, prior_hbm,
            bidx_v, eidx_v, widx_v, brow_v, erow_v, wrow_v, diff_v, pp_v,
            sem):
    wid = lax.axis_index("s") * NC + lax.axis_index("c")

    pltpu.sync_copy(bond_hbm.at[wid], bidx_v)
    pltpu.sync_copy(bett_hbm.at[wid], eidx_v)
    pltpu.sync_copy(wors_hbm.at[wid], widx_v)

    copies = []
    for j in range(NCHUNK):
        sl = pl.ds(j * ICHUNK, ICHUNK)
        copies.append(pltpu.async_copy(table_hbm.at[bidx_v.at[j]],
                                       brow_v.at[sl], sem))
        copies.append(pltpu.async_copy(table_hbm.at[eidx_v.at[j]],
                                       erow_v.at[sl], sem))
        copies.append(pltpu.async_copy(table_hbm.at[widx_v.at[j]],
                                       wrow_v.at[sl], sem))
    for c in copies:
        c.wait()

    iota = lax.iota(jnp.int32, L)

    def group_body(g, pp):
        rows = g * L + iota
        acc = jnp.zeros((L,), jnp.float32)
        for f in range(NUM_FACTORS):
            cols = jnp.full((L,), f, jnp.int32)
            b = plsc.load_gather(brow_v, [rows, cols])
            e = plsc.load_gather(erow_v, [rows, cols])
            w = plsc.load_gather(wrow_v, [rows, cols])
            acc = acc + b * (e - w)
            pp = pp + b * b + e * e + w * w
        diff_v[pl.ds(g * L, L)] = acc
        return pp

    pp = lax.fori_loop(0, NGROUPS, group_body, jnp.zeros((L,), jnp.float32))
    pp_v[...] = pp
    pltpu.sync_copy(diff_v, diff_hbm.at[wid])
    pltpu.sync_copy(pp_v, prior_hbm.at[wid])


def _tc_body(diff_ref, prior_ref, ll_ref, pr_ref, auc_ref):
    x = diff_ref[...]
    ls = jnp.minimum(x, 0.0) - jnp.log1p(jnp.exp(-jnp.abs(x)))
    inv_b = jnp.float32(1.0 / BATCH)
    ll_ref[0, 0] = jnp.sum(ls) * inv_b
    pr_ref[0, 0] = jnp.float32(REG) * jnp.sum(prior_ref[...])
    auc_ref[0, 0] = jnp.sum(jnp.where(x > 0, 1.0, 0.0)) * inv_b


_tc_epilogue = pl.pallas_call(
    _tc_body,
    out_shape=(
        jax.ShapeDtypeStruct((1, 1), jnp.float32),
        jax.ShapeDtypeStruct((1, 1), jnp.float32),
        jax.ShapeDtypeStruct((1, 1), jnp.float32),
    ),
    out_specs=(
        pl.BlockSpec(memory_space=pltpu.SMEM),
        pl.BlockSpec(memory_space=pltpu.SMEM),
        pl.BlockSpec(memory_space=pltpu.SMEM),
    ),
)


@jax.jit
def kernel(rankings, table):
    r = rankings.astype(jnp.int32)
    bonds = r[:, 0].reshape(NW, NCHUNK, ICHUNK)
    better = r[:, 1].reshape(NW, NCHUNK, ICHUNK)
    worse = r[:, 2].reshape(NW, NCHUNK, ICHUNK)
    diff, prior = _sc_bpr(bonds, better, worse, table)
    ll, pr, auc = _tc_epilogue(diff.reshape(128, 128), prior)
    return ll[0, 0], pr[0, 0], auc[0, 0]
